# baseline (speedup 1.0000x reference)
import functools

import jax
import jax.numpy as jnp
from jax import lax
from jax.experimental import pallas as pl
from jax.experimental.pallas import tpu as pltpu
from jax.experimental.pallas import tpu_sc as plsc

_NB = 1000
_RB = 256
_C = 128


def _gelu_exact(x):
    return 0.5 * x * (1.0 + jax.lax.erf(x * 0.7071067811865475))


def _pre_kernel(mem_ref, feat_ref, gene_ref, mw1_ref, mb1_ref, mw2_ref,
                ones_ref, out_ref):
    nb = mem_ref.shape[0]
    g = gene_ref[...].reshape(nb * 8, 50)
    m = _gelu_exact(jnp.dot(g, mw1_ref[...],
                            preferred_element_type=jnp.float32) + mb1_ref[...])
    t8 = jnp.dot(m, mw2_ref[...], preferred_element_type=jnp.float32)
    a = jnp.dot(jnp.abs(g), ones_ref[...],
                preferred_element_type=jnp.float32)
    aug = jnp.concatenate([t8, a], axis=1)
    red = jnp.sum(aug.reshape(nb, 8, 51), axis=1)
    out_ref[:, :172] = mem_ref[...] + feat_ref[...]
    out_ref[:, 172:223] = red
    out_ref[:, 223:] = jnp.zeros((nb, 33), jnp.float32)


def _run_pre(memory, node_features, node_gene, merge_w1, merge_b1, merge_w2,
             interpret=False):
    N = memory.shape[0]
    grid = (N // _NB,)

    def w_spec(a):
        return pl.BlockSpec(a.shape, lambda i: (0,) * a.ndim)

    mb1 = merge_b1.reshape(1, -1)
    ones = jnp.ones((50, 1), jnp.float32)
    return pl.pallas_call(
        _pre_kernel,
        grid=grid,
        in_specs=[pl.BlockSpec((_NB, 172), lambda i: (i, 0)),
                  pl.BlockSpec((_NB, 172), lambda i: (i, 0)),
                  pl.BlockSpec((_NB, 8, 50), lambda i: (i, 0, 0)),
                  w_spec(merge_w1), w_spec(mb1), w_spec(merge_w2),
                  w_spec(ones)],
        out_specs=pl.BlockSpec((_NB, 256), lambda i: (i, 0)),
        out_shape=jax.ShapeDtypeStruct((N, 256), jnp.float32),
        interpret=interpret,
    )(memory, node_features, node_gene, merge_w1, mb1, merge_w2, ones)


def _sc_gather(table, idx):
    info = plsc.get_sparse_core_info()
    nw = info.num_cores * info.num_subcores
    R = idx.shape[0]
    per_w = R // nw
    n_chunks = per_w // _C
    mesh = plsc.VectorSubcoreMesh(core_axis_name="c", subcore_axis_name="s")

    @functools.partial(
        pl.kernel, mesh=mesh,
        out_type=jax.ShapeDtypeStruct((R, 256), jnp.float32),
        scratch_types=[pltpu.VMEM((_C,), jnp.int32),
                       pltpu.VMEM((_C, 256), jnp.float32),
                       pltpu.SemaphoreType.DMA],
        compiler_params=pltpu.CompilerParams(use_tc_tiling_on_sc=True),
    )
    def gather_k(table_hbm, idx_hbm, out_hbm, idx_v, rows_v, sem):
        wid = lax.axis_index("s") * info.num_cores + lax.axis_index("c")
        base = wid * per_w

        def body(g, carry):
            off = base + g * _C
            pltpu.sync_copy(idx_hbm.at[pl.ds(off, _C)], idx_v)
            pltpu.async_copy(table_hbm.at[idx_v], rows_v, sem).wait()
            pltpu.sync_copy(rows_v, out_hbm.at[pl.ds(off, _C)])
            return carry

        lax.fori_loop(0, n_chunks, body, 0)

    return gather_k(table, idx)


def _chain_kernel(row_ref,
                  mb2_ref,
                  ww1_ref, wb1_ref, ww2_ref, wb2_ref,
                  pc_ref, ps_ref, p2_ref,
                  out_ref):
    taw = row_ref[:, 172:222]
    mask = jnp.sign(row_ref[:, 222:223])
    temp0 = (taw + 8.0 * mb2_ref[...]) * mask
    r = (jnp.dot(_gelu_exact(jnp.dot(temp0, ww1_ref[...],
                                     preferred_element_type=jnp.float32)
                             + wb1_ref[...]),
                 ww2_ref[...], preferred_element_type=jnp.float32)
         + wb2_ref[...])
    fc = jnp.cos(r) * 0.05
    fs = jnp.sin(r) * 0.05
    t = (jnp.dot(fc, pc_ref[...], preferred_element_type=jnp.float32)
         + jnp.dot(fs, ps_ref[...], preferred_element_type=jnp.float32))
    emb = jnp.dot(_gelu_exact(t), p2_ref[...],
                  preferred_element_type=jnp.float32)
    out_ref[:, :172] = row_ref[:, :172]
    out_ref[:, 172:] = emb


def _run_chain(rows,
               merge_b2,
               wr_w1, wr_b1, wr_w2, wr_b2, pe_w1, pe_w2, interpret=False):
    R = rows.shape[0]
    grid = (R // _RB,)

    def w_spec(a):
        return pl.BlockSpec(a.shape, lambda i: (0,) * a.ndim)

    pe_cos = pe_w1[:200, :]
    pe_sin = pe_w1[200:, :]
    weights = [merge_b2.reshape(1, -1),
               wr_w1, wr_b1.reshape(1, -1),
               wr_w2, wr_b2.reshape(1, -1), pe_cos, pe_sin, pe_w2]
    return pl.pallas_call(
        _chain_kernel,
        grid=grid,
        in_specs=[pl.BlockSpec((_RB, 256), lambda i: (i, 0))]
        + [w_spec(w) for w in weights],
        out_specs=pl.BlockSpec((_RB, 572), lambda i: (i, 0)),
        out_shape=jax.ShapeDtypeStruct((R, 572), jnp.float32),
        interpret=interpret,
    )(rows, *weights)


def kernel(node_features, memory, node_gene, source_nodes, neighbors,
           merge_w1, merge_b1, merge_w2, merge_b2,
           wr_w1, wr_b1, wr_w2, wr_b2, pe_w1, pe_w2):
    B, Nn = neighbors.shape
    idx = jnp.concatenate([source_nodes[:, None], neighbors],
                          axis=1).reshape(-1).astype(jnp.int32)
    table = _run_pre(memory, node_features, node_gene, merge_w1, merge_b1,
                     merge_w2)
    R = idx.shape[0]
    s = R // 3
    outs = []
    for lo in range(0, R, s):
        rows = _sc_gather(table, lax.dynamic_slice_in_dim(idx, lo, s))
        outs.append(_run_chain(rows, merge_b2,
                               wr_w1, wr_b1, wr_w2, wr_b2, pe_w1, pe_w2))
    return jnp.concatenate(outs, axis=0).reshape(B, 1 + Nn, 572)


# speedup vs baseline: 1.1807x; 1.1807x over previous
import functools

import jax
import jax.numpy as jnp
from jax import lax
from jax.experimental import pallas as pl
from jax.experimental.pallas import tpu as pltpu
from jax.experimental.pallas import tpu_sc as plsc

_NB = 1000
_RB = 336
_C = 128


def _gelu_exact(x):
    return 0.5 * x * (1.0 + jax.lax.erf(x * 0.7071067811865475))


def _pre_kernel(mem_ref, feat_ref, gene_ref, mw1_ref, mb1_ref, mw2_ref,
                ones_ref, out_ref):
    nb = mem_ref.shape[0]
    g = gene_ref[...].reshape(nb * 8, 50)
    m = _gelu_exact(jnp.dot(g, mw1_ref[...],
                            preferred_element_type=jnp.float32) + mb1_ref[...])
    t8 = jnp.dot(m, mw2_ref[...], preferred_element_type=jnp.float32)
    a = jnp.dot(jnp.abs(g), ones_ref[...],
                preferred_element_type=jnp.float32)
    aug = jnp.concatenate([t8, a], axis=1)
    red = jnp.sum(aug.reshape(nb, 8, 51), axis=1)
    out_ref[:, :172] = mem_ref[...] + feat_ref[...]
    out_ref[:, 172:223] = red
    out_ref[:, 223:] = jnp.zeros((nb, 33), jnp.float32)


def _run_pre(memory, node_features, node_gene, merge_w1, merge_b1, merge_w2,
             interpret=False):
    N = memory.shape[0]
    grid = (N // _NB,)

    def w_spec(a):
        return pl.BlockSpec(a.shape, lambda i: (0,) * a.ndim)

    mb1 = merge_b1.reshape(1, -1)
    ones = jnp.ones((50, 1), jnp.float32)
    return pl.pallas_call(
        _pre_kernel,
        grid=grid,
        in_specs=[pl.BlockSpec((_NB, 172), lambda i: (i, 0)),
                  pl.BlockSpec((_NB, 172), lambda i: (i, 0)),
                  pl.BlockSpec((_NB, 8, 50), lambda i: (i, 0, 0)),
                  w_spec(merge_w1), w_spec(mb1), w_spec(merge_w2),
                  w_spec(ones)],
        out_specs=pl.BlockSpec((_NB, 256), lambda i: (i, 0)),
        out_shape=jax.ShapeDtypeStruct((N, 256), jnp.float32),
        interpret=interpret,
    )(memory, node_features, node_gene, merge_w1, mb1, merge_w2, ones)


def _sc_gather(table, idx):
    info = plsc.get_sparse_core_info()
    nw = info.num_cores * info.num_subcores
    R = idx.shape[0]
    per_w = R // nw
    n_chunks = per_w // _C
    mesh = plsc.VectorSubcoreMesh(core_axis_name="c", subcore_axis_name="s")

    @functools.partial(
        pl.kernel, mesh=mesh,
        out_type=jax.ShapeDtypeStruct((R, 256), jnp.float32),
        scratch_types=[pltpu.VMEM((_C,), jnp.int32),
                       pltpu.VMEM((_C, 256), jnp.float32),
                       pltpu.SemaphoreType.DMA],
        compiler_params=pltpu.CompilerParams(use_tc_tiling_on_sc=True),
    )
    def gather_k(table_hbm, idx_hbm, out_hbm, idx_v, rows_v, sem):
        wid = lax.axis_index("s") * info.num_cores + lax.axis_index("c")
        base = wid * per_w

        def body(g, carry):
            off = base + g * _C
            pltpu.sync_copy(idx_hbm.at[pl.ds(off, _C)], idx_v)
            pltpu.async_copy(table_hbm.at[idx_v], rows_v, sem).wait()
            pltpu.sync_copy(rows_v, out_hbm.at[pl.ds(off, _C)])
            return carry

        lax.fori_loop(0, n_chunks, body, 0)

    return gather_k(table, idx)


def _chain_kernel(row_ref,
                  mb2_ref,
                  ww1_ref, wb1_ref, ww2_ref, wb2_ref,
                  pc_ref, ps_ref, p2_ref,
                  out_ref):
    taw = row_ref[:, 172:222]
    mask = jnp.sign(row_ref[:, 222:223])
    temp0 = (taw + 8.0 * mb2_ref[...]) * mask
    r = (jnp.dot(_gelu_exact(jnp.dot(temp0, ww1_ref[...],
                                     preferred_element_type=jnp.float32)
                             + wb1_ref[...]),
                 ww2_ref[...], preferred_element_type=jnp.float32)
         + wb2_ref[...])
    fc = jnp.cos(r) * 0.05
    fs = jnp.sin(r) * 0.05
    t = (jnp.dot(fc, pc_ref[...], preferred_element_type=jnp.float32)
         + jnp.dot(fs, ps_ref[...], preferred_element_type=jnp.float32))
    emb = jnp.dot(_gelu_exact(t), p2_ref[...],
                  preferred_element_type=jnp.float32)
    full = jnp.concatenate([row_ref[:, :172], emb], axis=1)
    for s in range(out_ref.shape[0]):
        out_ref[s] = full[s * 21:(s + 1) * 21, :]


def _run_chain(rows, B,
               merge_b2,
               wr_w1, wr_b1, wr_w2, wr_b2, pe_w1, pe_w2, interpret=False):
    R = rows.shape[0]
    nsrc = _RB // 21
    grid = (R // _RB,)

    def w_spec(a):
        return pl.BlockSpec(a.shape, lambda i: (0,) * a.ndim)

    pe_cos = pe_w1[:200, :]
    pe_sin = pe_w1[200:, :]
    weights = [merge_b2.reshape(1, -1),
               wr_w1, wr_b1.reshape(1, -1),
               wr_w2, wr_b2.reshape(1, -1), pe_cos, pe_sin, pe_w2]
    return pl.pallas_call(
        _chain_kernel,
        grid=grid,
        in_specs=[pl.BlockSpec((_RB, 256), lambda i: (i, 0))]
        + [w_spec(w) for w in weights],
        out_specs=pl.BlockSpec((nsrc, 21, 572), lambda i: (i, 0, 0)),
        out_shape=jax.ShapeDtypeStruct((B, 21, 572), jnp.float32),
        interpret=interpret,
    )(rows, *weights)


def kernel(node_features, memory, node_gene, source_nodes, neighbors,
           merge_w1, merge_b1, merge_w2, merge_b2,
           wr_w1, wr_b1, wr_w2, wr_b2, pe_w1, pe_w2):
    B, Nn = neighbors.shape
    idx = jnp.concatenate([source_nodes[:, None], neighbors],
                          axis=1).reshape(-1).astype(jnp.int32)
    table = _run_pre(memory, node_features, node_gene, merge_w1, merge_b1,
                     merge_w2)
    rows = _sc_gather(table, idx)
    return _run_chain(rows, B,
                      merge_b2,
                      wr_w1, wr_b1, wr_w2, wr_b2, pe_w1, pe_w2)
